# baseline (speedup 1.0000x reference)
import functools

import jax
import jax.numpy as jnp
from jax import lax
from jax.experimental import pallas as pl
from jax.experimental.pallas import tpu as pltpu
from jax.experimental.pallas import tpu_sc as plsc

L = 16


@functools.lru_cache(maxsize=None)
def _build(N, V, K):
    info = plsc.get_sparse_core_info()
    NC, NS = info.num_cores, info.num_subcores
    NW = NC * NS
    assert N % NW == 0
    n_rows = N // NW
    C = 3328
    assert n_rows % C == 0
    n_chunks = n_rows // C
    G = 128
    n_sub = C // G
    assert K == L

    mesh = plsc.VectorSubcoreMesh(core_axis_name="c", subcore_axis_name="s")

    @functools.partial(
        pl.kernel,
        mesh=mesh,
        out_type=jax.ShapeDtypeStruct((N, K), jnp.float32),
        compiler_params=pltpu.CompilerParams(use_tc_tiling_on_sc=False),
        scratch_types=[
            pltpu.VMEM((n_rows,), jnp.int32),
            pltpu.VMEM((n_rows,), jnp.float32),
            pltpu.VMEM((C, K), jnp.float32),
            pltpu.SemaphoreType.DMA,
        ],
    )
    def sc_kernel(idx_hbm, val_hbm, table_hbm, out_hbm, idx_v, val_v, rows_v, sem):
        wid = lax.axis_index("s") * NC + lax.axis_index("c")
        base = wid * n_rows
        pltpu.sync_copy(idx_hbm.at[pl.ds(base, n_rows)], idx_v)
        pltpu.sync_copy(val_hbm.at[pl.ds(base, n_rows)], val_v)
        for c in range(n_chunks):
            coff = c * C
            cps = [
                pltpu.async_copy(
                    table_hbm.at[idx_v.at[pl.ds(coff + g * G, G)]],
                    rows_v.at[pl.ds(g * G, G)],
                    sem,
                )
                for g in range(n_sub)
            ]
            for cp in cps:
                cp.wait()

            def mul_body(i, carry, coff=coff):
                r0 = i * L
                val16 = val_v[pl.ds(coff + r0, L)]
                for j in range(L):
                    r = r0 + j
                    vj = jnp.full((L,), val16[j])
                    rows_v[r, :] = rows_v[r, :] * vj
                return carry

            lax.fori_loop(0, C // L, mul_body, 0)
            pltpu.sync_copy(rows_v, out_hbm.at[pl.ds(base + coff, C)])

    return sc_kernel


def kernel(nonzero_index, nonzero_value, table):
    B, F = nonzero_index.shape
    V, K = table.shape
    N = B * F
    idx = nonzero_index.reshape(N).astype(jnp.int32)
    val = nonzero_value.reshape(N)
    out = _build(N, V, K)(idx, val, table)
    return out.reshape(B, F, K)
